# baseline (speedup 1.0000x reference)
import jax
import jax.numpy as jnp
from jax import lax
from jax.experimental import pallas as pl
from jax.experimental.pallas import tpu as pltpu
from jax.experimental.pallas import tpu_sc as plsc

NK = 1024
D = 64
DP = 128
B = 16
PIX = 1024
N = B * PIX
NW = 32
RPW = N // NW
GSZ = 128
G = RPW // GSZ
RPT = NK // 16

DECAY = 0.99
EPS = 1e-5
CC = 0.25


def _argmin_body(x_ref, w_ref, idx_ref, xt_ref):
    X = x_ref[0]
    Wm = w_ref[...]
    S = lax.dot_general(Wm, X, (((1,), (0,)), ((), ())),
                        preferred_element_type=jnp.float32)
    w2 = jnp.sum(Wm * Wm, axis=1, keepdims=True)
    x2 = jnp.sum(X * X, axis=0, keepdims=True)
    dist = (x2 + w2) - 2.0 * S
    dmin = jnp.min(dist, axis=0, keepdims=True)
    rows = lax.broadcasted_iota(jnp.int32, (NK, PIX), 0)
    idx = jnp.min(jnp.where(dist == dmin, rows, NK), axis=0)
    idx_ref[0, 0, :] = idx
    ey = (lax.broadcasted_iota(jnp.int32, (D, DP), 0)
          == lax.broadcasted_iota(jnp.int32, (D, DP), 1)).astype(jnp.float32)
    onecol = (lax.broadcasted_iota(jnp.int32, (PIX, DP), 1) == D)
    xt_ref[0] = (lax.dot_general(X, ey, (((0,), (0,)), ((), ())),
                                 preferred_element_type=jnp.float32)
                 + onecol.astype(jnp.float32))


def _stage_a(x3, W):
    return pl.pallas_call(
        _argmin_body,
        grid=(B,),
        in_specs=[pl.BlockSpec((1, D, PIX), lambda i: (i, 0, 0)),
                  pl.BlockSpec((NK, D), lambda i: (0, 0))],
        out_specs=[pl.BlockSpec((1, 1, PIX), lambda i: (i, 0, 0)),
                   pl.BlockSpec((1, PIX, DP), lambda i: (i, 0, 0))],
        out_shape=[jax.ShapeDtypeStruct((B, 1, PIX), jnp.int32),
                   jax.ShapeDtypeStruct((B, PIX, DP), jnp.float32)],
    )(x3, W)


def _scatter_body(x_hbm, idx_hbm, zdw_hbm, dwp_hbm,
                  idx_v, rows_v, dw_sh):
    c = lax.axis_index("c")
    s = lax.axis_index("s")
    w = s * 2 + c
    pltpu.sync_copy(idx_hbm.at[w], idx_v)
    pltpu.sync_copy(x_hbm.at[pl.ds(w * RPW, RPW)], rows_v)

    @pl.when(s == 0)
    def _():
        pltpu.sync_copy(zdw_hbm, dw_sh)

    plsc.subcore_barrier()
    for g in range(G):
        pltpu.sync_copy(rows_v.at[pl.ds(g * GSZ, GSZ)],
                        dw_sh.at[idx_v.at[g]], add=True)
    plsc.subcore_barrier()
    pltpu.sync_copy(dw_sh.at[pl.ds(s * RPT, RPT)],
                    dwp_hbm.at[c, pl.ds(s * RPT, RPT)])


def _stage_b(x_flat, idx_g, zdw):
    mesh = plsc.VectorSubcoreMesh(core_axis_name="c", subcore_axis_name="s")
    f = pl.kernel(
        _scatter_body,
        out_type=jax.ShapeDtypeStruct((2, NK, DP), jnp.float32),
        mesh=mesh,
        scratch_types=[pltpu.VMEM((G, GSZ), jnp.int32),
                       pltpu.VMEM((RPW, DP), jnp.float32),
                       pltpu.VMEM_SHARED((NK, DP), jnp.float32)],
    )
    return f(x_flat, idx_g, zdw)


def _ema_body(dwp_ref, ecs_ref, emaw_ref, wnew_ref):
    acc = dwp_ref[0] + dwp_ref[1]
    dw = acc[:, :D]
    counts = acc[:, D:D + 1]
    ncs = ecs_ref[...] * DECAY + (1.0 - DECAY) * counts
    n = jnp.sum(ncs)
    cs = (ncs + EPS) / (n + NK * EPS) * n
    wnew = (emaw_ref[...] * DECAY + (1.0 - DECAY) * dw) / cs
    ey = (lax.broadcasted_iota(jnp.int32, (D, DP), 0)
          == lax.broadcasted_iota(jnp.int32, (D, DP), 1)).astype(jnp.float32)
    wnew_ref[...] = lax.dot_general(wnew, ey, (((1,), (0,)), ((), ())),
                                    preferred_element_type=jnp.float32)


def _stage_c(dwp, ecs2, ema_w):
    return pl.pallas_call(
        _ema_body,
        out_shape=jax.ShapeDtypeStruct((NK, DP), jnp.float32),
    )(dwp, ecs2, ema_w)


def _gather_body(wnew_hbm, idx_hbm, q_hbm, idx_v, rows_v, wnew_sh, sem):
    c = lax.axis_index("c")
    s = lax.axis_index("s")
    w = s * 2 + c
    pltpu.sync_copy(idx_hbm.at[w], idx_v)
    pltpu.sync_copy(wnew_hbm.at[pl.ds(s * RPT, RPT)],
                    wnew_sh.at[pl.ds(s * RPT, RPT)])
    plsc.subcore_barrier()
    for g in range(G):
        pltpu.async_copy(wnew_sh.at[idx_v.at[g]],
                         rows_v.at[pl.ds(g * GSZ, GSZ)], sem).wait()
    pltpu.sync_copy(rows_v, q_hbm.at[pl.ds(w * RPW, RPW)])


def _stage_d(wnew, idx_g):
    mesh = plsc.VectorSubcoreMesh(core_axis_name="c", subcore_axis_name="s")
    f = pl.kernel(
        _gather_body,
        out_type=jax.ShapeDtypeStruct((N, DP), jnp.float32),
        mesh=mesh,
        scratch_types=[pltpu.VMEM((G, GSZ), jnp.int32),
                       pltpu.VMEM((RPW, DP), jnp.float32),
                       pltpu.VMEM_SHARED((NK, DP), jnp.float32),
                       pltpu.SemaphoreType.DMA],
    )
    return f(wnew, idx_g)


def _finish_body(q_ref, xt_ref, qout_ref, loss_ref, acc_ref):
    i = pl.program_id(0)
    q = q_ref[0][:, :D]
    x = xt_ref[0][:, :D]
    dd = q - x
    part = jnp.sum(dd * dd)

    @pl.when(i == 0)
    def _():
        acc_ref[0, 0] = 0.0

    acc_ref[0, 0] += part
    ey = (lax.broadcasted_iota(jnp.int32, (D, DP), 0)
          == lax.broadcasted_iota(jnp.int32, (D, DP), 1)).astype(jnp.float32)
    qout_ref[0] = lax.dot_general(ey, q_ref[0], (((1,), (1,)), ((), ())),
                                  preferred_element_type=jnp.float32)

    @pl.when(i == B - 1)
    def _():
        loss_ref[0, 0] = acc_ref[0, 0] * (CC / (N * D))


def _stage_e(q3, xt):
    return pl.pallas_call(
        _finish_body,
        grid=(B,),
        in_specs=[pl.BlockSpec((1, PIX, DP), lambda i: (i, 0, 0)),
                  pl.BlockSpec((1, PIX, DP), lambda i: (i, 0, 0))],
        out_specs=[pl.BlockSpec((1, D, PIX), lambda i: (i, 0, 0)),
                   pl.BlockSpec((1, 1), lambda i: (0, 0),
                                memory_space=pltpu.SMEM)],
        out_shape=[jax.ShapeDtypeStruct((B, D, PIX), jnp.float32),
                   jax.ShapeDtypeStruct((1, 1), jnp.float32)],
        scratch_shapes=[pltpu.SMEM((1, 1), jnp.float32)],
    )(q3, xt)


def kernel(inputs, W, ema_cluster_size, ema_w):
    x3 = inputs.reshape(B, D, PIX)
    idx3, xt = _stage_a(x3, W)
    idx_g = idx3.reshape(NW, G, GSZ)
    x_flat = xt.reshape(N, DP)
    zdw = jnp.zeros((NK, DP), jnp.float32)
    dwp = _stage_b(x_flat, idx_g, zdw)
    wnew_p = _stage_c(dwp, ema_cluster_size.reshape(NK, 1), ema_w)
    qp = _stage_d(wnew_p, idx_g)
    qout, loss2 = _stage_e(qp.reshape(B, PIX, DP), xt)
    return (qout.reshape(B, D, 32, 32), loss2.reshape(()),
            idx3.reshape(B, 32, 32))
